# baseline (speedup 1.0000x reference)
import functools

import jax
import jax.numpy as jnp
from jax import lax
from jax.experimental import pallas as pl
from jax.experimental.pallas import tpu as pltpu
from jax.experimental.pallas import tpu_sc as plsc

N_USERS = 5000
M_ITEMS = 5000
N_NODES = N_USERS + M_ITEMS
N_EDGES = 320000
D = 128
DH = 64
PW = 32
N_LAYERS = 3

NC = 2
NS = 16

CHUNK = 128
N_CHUNKS = 160
E_PER_TILE = CHUNK * N_CHUNKS
E_PAD = E_PER_TILE * NS
NCH_TOT = E_PAD // CHUNK
ZCH = 80
N_ZCH = N_NODES // ZCH


def _tc_prologue(user_emb, item_emb, creator_feat, item_feat, Wc, bc, Wi, bi, lw):
    def body(u_ref, i_ref, cf_ref, if_ref, wc_ref, bc_ref, wi_ref, bi_ref,
             lw_ref, emb_ref, acc_ref):
        u = u_ref[...] + jnp.dot(cf_ref[...], wc_ref[...],
                                 preferred_element_type=jnp.float32)
        u = u + bc_ref[...][None, :]
        it = i_ref[...] + jnp.dot(if_ref[...], wi_ref[...],
                                  preferred_element_type=jnp.float32)
        it = it + bi_ref[...][None, :]
        w0 = lw_ref[0]
        full = jnp.concatenate([u, it], axis=0)
        for cc in range(NC):
            half = full[:, cc * DH:(cc + 1) * DH]
            acc_ref[cc] = half * w0
            lo = lax.bitcast_convert_type(half[:, :PW], jnp.int32)
            hi = lax.bitcast_convert_type(half[:, PW:], jnp.int32)
            lor = lo + jnp.int32(0x7FFF) + ((lo >> 16) & 1)
            hir = hi + jnp.int32(0x7FFF) + ((hi >> 16) & 1)
            emb_ref[cc] = (lax.shift_right_logical(lor, 16)
                           | (hir & jnp.int32(-65536)))

    return pl.pallas_call(
        body,
        out_shape=(
            jax.ShapeDtypeStruct((NC, N_NODES, PW), jnp.int32),
            jax.ShapeDtypeStruct((NC, N_NODES, DH), jnp.float32),
        ),
        in_specs=[
            pl.BlockSpec(memory_space=pltpu.VMEM),
            pl.BlockSpec(memory_space=pltpu.VMEM),
            pl.BlockSpec(memory_space=pltpu.VMEM),
            pl.BlockSpec(memory_space=pltpu.VMEM),
            pl.BlockSpec(memory_space=pltpu.VMEM),
            pl.BlockSpec(memory_space=pltpu.VMEM),
            pl.BlockSpec(memory_space=pltpu.VMEM),
            pl.BlockSpec(memory_space=pltpu.VMEM),
            pl.BlockSpec(memory_space=pltpu.SMEM),
        ],
        out_specs=(
            pl.BlockSpec(memory_space=pltpu.VMEM),
            pl.BlockSpec(memory_space=pltpu.VMEM),
        ),
    )(user_emb, item_emb, creator_feat, item_feat, Wc, bc, Wi, bi, lw)


def _lane_broadcast(vvec, j):
    bidx = jnp.broadcast_to(j, (16,)).astype(jnp.int32)
    dnums = lax.GatherDimensionNumbers(
        offset_dims=(), collapsed_slice_dims=(0,), start_index_map=(0,))
    return lax.gather(vvec, bidx[:, None], dnums, slice_sizes=(1,),
                      mode=lax.GatherScatterMode.PROMISE_IN_BOUNDS)


_MASK_HI = -65536


def _sc_layer(emb_flat, cols2, rows2, vals2, acc, wvec16):
    mesh = plsc.VectorSubcoreMesh(core_axis_name="c", subcore_axis_name="s")

    @functools.partial(
        pl.kernel,
        mesh=mesh,
        compiler_params=pltpu.CompilerParams(use_tc_tiling_on_sc=False),
        out_type=(
            jax.ShapeDtypeStruct((NC, N_NODES, PW), jnp.int32),
            jax.ShapeDtypeStruct((NC, N_NODES, DH), jnp.float32),
        ),
        scratch_types=[
            pltpu.VMEM_SHARED((N_NODES, DH), jnp.float32),
            pltpu.VMEM((N_CHUNKS, CHUNK), jnp.int32),
            pltpu.VMEM((CHUNK, 16), jnp.float32),
            pltpu.VMEM((CHUNK, 16), jnp.float32),
            pltpu.VMEM((CHUNK,), jnp.int32),
            pltpu.VMEM((CHUNK,), jnp.int32),
            pltpu.VMEM((CHUNK,), jnp.int32),
            pltpu.VMEM((CHUNK,), jnp.int32),
            pltpu.VMEM((CHUNK, PW), jnp.int32),
            pltpu.VMEM((CHUNK, PW), jnp.int32),
            pltpu.VMEM((CHUNK, PW), jnp.int32),
            pltpu.VMEM((CHUNK, PW), jnp.int32),
            pltpu.VMEM((CHUNK, DH), jnp.float32),
            pltpu.VMEM((CHUNK, DH), jnp.float32),
            pltpu.VMEM((16,), jnp.float32),
            pltpu.SemaphoreType.DMA,
            pltpu.SemaphoreType.DMA,
            pltpu.SemaphoreType.DMA,
            pltpu.SemaphoreType.DMA,
            pltpu.SemaphoreType.DMA,
            pltpu.SemaphoreType.DMA,
            pltpu.SemaphoreType.DMA,
            pltpu.SemaphoreType.DMA,
            pltpu.SemaphoreType.DMA,
            pltpu.SemaphoreType.DMA,
            pltpu.SemaphoreType.DMA,
            pltpu.SemaphoreType.DMA,
        ],
    )
    def run(emb_hbm, cols_hbm, rows_hbm, vals_hbm, acc_hbm, w_hbm,
            embout_hbm, accout_hbm,
            acc_sh, rows_t, vb0, vb1, c0, c1, c2, c3, g0, g1, g2, g3,
            s0, s1, wbuf,
            is0, is1, is2, is3, gs0, gs1, gs2, gs3, ss0, ss1, vs0, vs1):
        c = lax.axis_index("c")
        s = lax.axis_index("s")
        colb = (c0, c1, c2, c3)
        gbufs = (g0, g1, g2, g3)
        sbufs = (s0, s1)
        isems = (is0, is1, is2, is3)
        gsems = (gs0, gs1, gs2, gs3)
        ssems = (ss0, ss1)
        vbufs = (vb0, vb1)
        vsems = (vs0, vs1)
        tb = s * N_CHUNKS

        def zrow(r, _):
            def zcol(d, _):
                s0[r, pl.ds(d * 16, 16)] = jnp.zeros((16,), jnp.float32)
                return 0
            return lax.fori_loop(0, DH // 16, zcol, 0)
        lax.fori_loop(0, ZCH, zrow, 0)

        def zcopy(k, _):
            idx = s + k * NS

            @pl.when(idx < N_ZCH)
            def _():
                pltpu.sync_copy(s0.at[pl.ds(0, ZCH)], acc_sh.at[pl.ds(idx * ZCH, ZCH)])
            return 0
        lax.fori_loop(0, (N_ZCH + NS - 1) // NS, zcopy, 0)

        pltpu.sync_copy(rows_hbm.at[pl.ds(tb, N_CHUNKS)], rows_t)
        pltpu.sync_copy(w_hbm, wbuf)
        for q in range(2):
            pltpu.async_copy(vals_hbm.at[tb + q], vbufs[q], vsems[q])
        for q in range(4):
            pltpu.async_copy(cols_hbm.at[c, tb + q], colb[q], isems[q])
        plsc.subcore_barrier()

        for q in range(2):
            pltpu.make_async_copy(cols_hbm.at[c, tb + q], colb[q],
                                  isems[q]).wait()
            pltpu.async_copy(emb_hbm.at[colb[q]], gbufs[q], gsems[q])

        def step_body(st, _):
            for k in range(4):
                i = st * 4 + k
                k2 = (k + 2) % 4
                b = k % 2
                gb, sb = gbufs[k], sbufs[b]

                pltpu.make_async_copy(emb_hbm.at[colb[k]], gb, gsems[k]).wait()

                @pl.when(i >= 2)
                def _():
                    pltpu.make_async_copy(
                        sb, acc_sh.at[rows_t.at[i - 2]], ssems[b]).wait()

                @pl.when(i + 4 < N_CHUNKS)
                def _():
                    pltpu.async_copy(cols_hbm.at[c, tb + i + 4], colb[k],
                                     isems[k])

                @pl.when(i + 2 < N_CHUNKS)
                def _():
                    pltpu.make_async_copy(cols_hbm.at[c, tb + i + 2],
                                          colb[k2], isems[k2]).wait()
                    pltpu.async_copy(emb_hbm.at[colb[k2]], gbufs[k2],
                                     gsems[k2])

                vb = vbufs[b]
                pltpu.make_async_copy(vals_hbm.at[tb + i], vb, vsems[b]).wait()

                def scale_group(g, _):
                    e0 = g * 16
                    for j in range(16):
                        bval = vb[e0 + j, pl.ds(0, 16)]
                        for kk in range(PW // 16):
                            ivec = gb[e0 + j, pl.ds(kk * 16, 16)]
                            flo = lax.bitcast_convert_type(
                                lax.shift_left(ivec, 16), jnp.float32)
                            fhi = lax.bitcast_convert_type(
                                ivec & _MASK_HI, jnp.float32)
                            sb[e0 + j, pl.ds(kk * 16, 16)] = flo * bval
                            sb[e0 + j, pl.ds(PW + kk * 16, 16)] = fhi * bval
                    return 0
                lax.fori_loop(0, CHUNK // 16, scale_group, 0)

                @pl.when(i + 2 < N_CHUNKS)
                def _():
                    pltpu.async_copy(vals_hbm.at[tb + i + 2], vb, vsems[b])

                pltpu.async_copy(sb, acc_sh.at[rows_t.at[i]], ssems[b],
                                 add=True)
            return 0
        lax.fori_loop(0, N_CHUNKS // 4, step_body, 0)
        pltpu.make_async_copy(s0, acc_sh.at[rows_t.at[N_CHUNKS - 2]],
                              ssems[0]).wait()
        pltpu.make_async_copy(s1, acc_sh.at[rows_t.at[N_CHUNKS - 1]],
                              ssems[1]).wait()
        plsc.subcore_barrier()

        wv = wbuf[...]

        def ecopy(k, _):
            idx = s + k * NS

            @pl.when(idx < N_ZCH)
            def _():
                r0 = idx * ZCH
                pltpu.sync_copy(acc_sh.at[pl.ds(r0, ZCH)], s0.at[pl.ds(0, ZCH)])
                pltpu.sync_copy(acc_hbm.at[c, pl.ds(r0, ZCH)], s1.at[pl.ds(0, ZCH)])

                def erow(r, _):
                    for kk in range(DH // 16):
                        sl = pl.ds(kk * 16, 16)
                        s1[r, sl] = s1[r, sl] + s0[r, sl] * wv
                    for kk in range(PW // 16):
                        lo = lax.bitcast_convert_type(
                            s0[r, pl.ds(kk * 16, 16)], jnp.int32)
                        hi = lax.bitcast_convert_type(
                            s0[r, pl.ds(PW + kk * 16, 16)], jnp.int32)
                        lor = lo + jnp.int32(0x7FFF) + ((lo >> 16) & 1)
                        hir = hi + jnp.int32(0x7FFF) + ((hi >> 16) & 1)
                        g0[r, pl.ds(kk * 16, 16)] = (
                            lax.shift_right_logical(lor, 16)
                            | (hir & _MASK_HI))
                    return 0
                lax.fori_loop(0, ZCH, erow, 0)

                pltpu.sync_copy(s1.at[pl.ds(0, ZCH)], accout_hbm.at[c, pl.ds(r0, ZCH)])
                pltpu.sync_copy(g0.at[pl.ds(0, ZCH)], embout_hbm.at[c, pl.ds(r0, ZCH)])
            return 0
        lax.fori_loop(0, (N_ZCH + NS - 1) // NS, ecopy, 0)

    return run(emb_flat, cols2, rows2, vals2, acc, wvec16)


def kernel(user_emb, item_emb, creator_feat, item_feat, Wc, bc, Wi, bi,
           adj_values, layer_weights, adj_indices):
    rows = adj_indices[0]
    cols = adj_indices[1]
    pad = E_PAD - N_EDGES
    pad_idx = (jnp.arange(pad, dtype=jnp.int32) * 13) % N_NODES
    rows2 = jnp.concatenate([rows, pad_idx]).reshape(NCH_TOT, CHUNK)
    cols_flat = jnp.concatenate([cols, pad_idx]).reshape(NCH_TOT, CHUNK)
    cols2 = jnp.stack([cols_flat, cols_flat + N_NODES], axis=0)
    vals_flat = jnp.concatenate([adj_values, jnp.zeros((pad,), jnp.float32)])
    vals2 = jnp.broadcast_to(
        vals_flat[:, None], (E_PAD, 16)).reshape(NCH_TOT, CHUNK, 16)

    emb_p, acc = _tc_prologue(user_emb, item_emb, creator_feat, item_feat,
                              Wc, bc, Wi, bi, layer_weights)
    for l in range(1, N_LAYERS + 1):
        wvec16 = jnp.broadcast_to(layer_weights[l], (16,))
        emb_p, acc = _sc_layer(emb_p.reshape(NC * N_NODES, PW), cols2, rows2,
                               vals2, acc, wvec16)

    final = jnp.concatenate([acc[0], acc[1]], axis=1)
    return final[:N_USERS], final[N_USERS:]


# speedup vs baseline: 1.3838x; 1.3838x over previous
import functools

import jax
import jax.numpy as jnp
from jax import lax
from jax.experimental import pallas as pl
from jax.experimental.pallas import tpu as pltpu
from jax.experimental.pallas import tpu_sc as plsc

N_USERS = 5000
M_ITEMS = 5000
N_NODES = N_USERS + M_ITEMS
N_EDGES = 320000
D = 128
DH = 64
PW = 32
N_LAYERS = 3

NC = 2
NS = 16

CHUNK = 128
N_CHUNKS = 160
E_PER_TILE = CHUNK * N_CHUNKS
E_PAD = E_PER_TILE * NS
NCH_TOT = E_PAD // CHUNK
ZCH = 80
N_ZCH = N_NODES // ZCH


def _tc_prologue(user_emb, item_emb, creator_feat, item_feat, Wc, bc, Wi, bi, lw):
    def body(u_ref, i_ref, cf_ref, if_ref, wc_ref, bc_ref, wi_ref, bi_ref,
             lw_ref, emb_ref, acc_ref):
        u = u_ref[...] + jnp.dot(cf_ref[...], wc_ref[...],
                                 preferred_element_type=jnp.float32)
        u = u + bc_ref[...][None, :]
        it = i_ref[...] + jnp.dot(if_ref[...], wi_ref[...],
                                  preferred_element_type=jnp.float32)
        it = it + bi_ref[...][None, :]
        w0 = lw_ref[0]
        full = jnp.concatenate([u, it], axis=0)
        for cc in range(NC):
            half = full[:, cc * DH:(cc + 1) * DH]
            acc_ref[cc] = half * w0
            lo = lax.bitcast_convert_type(half[:, :PW], jnp.int32)
            hi = lax.bitcast_convert_type(half[:, PW:], jnp.int32)
            lor = lo + jnp.int32(0x7FFF) + ((lo >> 16) & 1)
            hir = hi + jnp.int32(0x7FFF) + ((hi >> 16) & 1)
            emb_ref[cc] = (lax.shift_right_logical(lor, 16)
                           | (hir & jnp.int32(-65536)))

    return pl.pallas_call(
        body,
        out_shape=(
            jax.ShapeDtypeStruct((NC, N_NODES, PW), jnp.int32),
            jax.ShapeDtypeStruct((NC, N_NODES, DH), jnp.float32),
        ),
        in_specs=[
            pl.BlockSpec(memory_space=pltpu.VMEM),
            pl.BlockSpec(memory_space=pltpu.VMEM),
            pl.BlockSpec(memory_space=pltpu.VMEM),
            pl.BlockSpec(memory_space=pltpu.VMEM),
            pl.BlockSpec(memory_space=pltpu.VMEM),
            pl.BlockSpec(memory_space=pltpu.VMEM),
            pl.BlockSpec(memory_space=pltpu.VMEM),
            pl.BlockSpec(memory_space=pltpu.VMEM),
            pl.BlockSpec(memory_space=pltpu.SMEM),
        ],
        out_specs=(
            pl.BlockSpec(memory_space=pltpu.VMEM),
            pl.BlockSpec(memory_space=pltpu.VMEM),
        ),
    )(user_emb, item_emb, creator_feat, item_feat, Wc, bc, Wi, bi, lw)


def _lane_broadcast(vvec, j):
    bidx = jnp.broadcast_to(j, (16,)).astype(jnp.int32)
    dnums = lax.GatherDimensionNumbers(
        offset_dims=(), collapsed_slice_dims=(0,), start_index_map=(0,))
    return lax.gather(vvec, bidx[:, None], dnums, slice_sizes=(1,),
                      mode=lax.GatherScatterMode.PROMISE_IN_BOUNDS)


_MASK_HI = -65536


def _sc_layer(emb_flat, cols2, rows2, vals2, acc, wvec16):
    mesh = plsc.VectorSubcoreMesh(core_axis_name="c", subcore_axis_name="s")

    @functools.partial(
        pl.kernel,
        mesh=mesh,
        compiler_params=pltpu.CompilerParams(use_tc_tiling_on_sc=False),
        out_type=(
            jax.ShapeDtypeStruct((NC, N_NODES, PW), jnp.int32),
            jax.ShapeDtypeStruct((NC, N_NODES, DH), jnp.float32),
        ),
        scratch_types=[
            pltpu.VMEM_SHARED((N_NODES, DH), jnp.float32),
            pltpu.VMEM((N_CHUNKS, CHUNK), jnp.int32),
            pltpu.VMEM((CHUNK, 16), jnp.float32),
            pltpu.VMEM((CHUNK, 16), jnp.float32),
            pltpu.VMEM((CHUNK,), jnp.int32),
            pltpu.VMEM((CHUNK,), jnp.int32),
            pltpu.VMEM((CHUNK,), jnp.int32),
            pltpu.VMEM((CHUNK,), jnp.int32),
            pltpu.VMEM((CHUNK, PW), jnp.int32),
            pltpu.VMEM((CHUNK, PW), jnp.int32),
            pltpu.VMEM((CHUNK, PW), jnp.int32),
            pltpu.VMEM((CHUNK, PW), jnp.int32),
            pltpu.VMEM((CHUNK, DH), jnp.float32),
            pltpu.VMEM((CHUNK, DH), jnp.float32),
            pltpu.VMEM((16,), jnp.float32),
            pltpu.SemaphoreType.DMA,
            pltpu.SemaphoreType.DMA,
            pltpu.SemaphoreType.DMA,
            pltpu.SemaphoreType.DMA,
            pltpu.SemaphoreType.DMA,
            pltpu.SemaphoreType.DMA,
            pltpu.SemaphoreType.DMA,
            pltpu.SemaphoreType.DMA,
            pltpu.SemaphoreType.DMA,
            pltpu.SemaphoreType.DMA,
            pltpu.SemaphoreType.DMA,
            pltpu.SemaphoreType.DMA,
        ],
    )
    def run(emb_hbm, cols_hbm, rows_hbm, vals_hbm, acc_hbm, w_hbm,
            embout_hbm, accout_hbm,
            acc_sh, rows_t, vb0, vb1, c0, c1, c2, c3, g0, g1, g2, g3,
            s0, s1, wbuf,
            is0, is1, is2, is3, gs0, gs1, gs2, gs3, ss0, ss1, vs0, vs1):
        c = lax.axis_index("c")
        s = lax.axis_index("s")
        colb = (c0, c1, c2, c3)
        gbufs = (g0, g1, g2, g3)
        sbufs = (s0, s1)
        isems = (is0, is1, is2, is3)
        gsems = (gs0, gs1, gs2, gs3)
        ssems = (ss0, ss1)
        vbufs = (vb0, vb1)
        vsems = (vs0, vs1)
        tb = s * N_CHUNKS

        def zrow(r, _):
            def zcol(d, _):
                s0[r, pl.ds(d * 16, 16)] = jnp.zeros((16,), jnp.float32)
                return 0
            return lax.fori_loop(0, DH // 16, zcol, 0)
        lax.fori_loop(0, ZCH, zrow, 0)

        def zcopy(k, _):
            idx = s + k * NS

            @pl.when(idx < N_ZCH)
            def _():
                pltpu.sync_copy(s0.at[pl.ds(0, ZCH)], acc_sh.at[pl.ds(idx * ZCH, ZCH)])
            return 0
        lax.fori_loop(0, (N_ZCH + NS - 1) // NS, zcopy, 0)

        pltpu.sync_copy(rows_hbm.at[pl.ds(tb, N_CHUNKS)], rows_t)
        pltpu.sync_copy(w_hbm, wbuf)
        for q in range(2):
            pltpu.async_copy(vals_hbm.at[tb + q], vbufs[q], vsems[q])
        for q in range(4):
            pltpu.async_copy(cols_hbm.at[c, tb + q], colb[q], isems[q])
        plsc.subcore_barrier()

        for q in range(2):
            pltpu.make_async_copy(cols_hbm.at[c, tb + q], colb[q],
                                  isems[q]).wait()
            pltpu.async_copy(emb_hbm.at[colb[q]], gbufs[q], gsems[q])

        def step_body(st, _):
            for k in range(4):
                i = st * 4 + k
                k2 = (k + 2) % 4
                b = k % 2
                gb, sb = gbufs[k], sbufs[b]

                pltpu.make_async_copy(emb_hbm.at[colb[k]], gb, gsems[k]).wait()

                @pl.when(i >= 2)
                def _():
                    pltpu.make_async_copy(
                        sb, acc_sh.at[rows_t.at[i - 2]], ssems[b]).wait()

                @pl.when(i + 4 < N_CHUNKS)
                def _():
                    pltpu.async_copy(cols_hbm.at[c, tb + i + 4], colb[k],
                                     isems[k])

                @pl.when(i + 2 < N_CHUNKS)
                def _():
                    pltpu.make_async_copy(cols_hbm.at[c, tb + i + 2],
                                          colb[k2], isems[k2]).wait()
                    pltpu.async_copy(emb_hbm.at[colb[k2]], gbufs[k2],
                                     gsems[k2])

                vb = vbufs[b]
                pltpu.make_async_copy(vals_hbm.at[tb + i], vb, vsems[b]).wait()

                for g in range(CHUNK // 16):
                    e0 = g * 16
                    for j in range(16):
                        bval = vb[e0 + j, pl.ds(0, 16)]
                        for kk in range(PW // 16):
                            ivec = gb[e0 + j, pl.ds(kk * 16, 16)]
                            flo = lax.bitcast_convert_type(
                                lax.shift_left(ivec, 16), jnp.float32)
                            fhi = lax.bitcast_convert_type(
                                ivec & _MASK_HI, jnp.float32)
                            sb[e0 + j, pl.ds(kk * 16, 16)] = flo * bval
                            sb[e0 + j, pl.ds(PW + kk * 16, 16)] = fhi * bval

                @pl.when(i + 2 < N_CHUNKS)
                def _():
                    pltpu.async_copy(vals_hbm.at[tb + i + 2], vb, vsems[b])

                pltpu.async_copy(sb, acc_sh.at[rows_t.at[i]], ssems[b],
                                 add=True)
            return 0
        lax.fori_loop(0, N_CHUNKS // 4, step_body, 0)
        pltpu.make_async_copy(s0, acc_sh.at[rows_t.at[N_CHUNKS - 2]],
                              ssems[0]).wait()
        pltpu.make_async_copy(s1, acc_sh.at[rows_t.at[N_CHUNKS - 1]],
                              ssems[1]).wait()
        plsc.subcore_barrier()

        wv = wbuf[...]

        def ecopy(k, _):
            idx = s + k * NS

            @pl.when(idx < N_ZCH)
            def _():
                r0 = idx * ZCH
                pltpu.sync_copy(acc_sh.at[pl.ds(r0, ZCH)], s0.at[pl.ds(0, ZCH)])
                pltpu.sync_copy(acc_hbm.at[c, pl.ds(r0, ZCH)], s1.at[pl.ds(0, ZCH)])

                def erow(r, _):
                    for kk in range(DH // 16):
                        sl = pl.ds(kk * 16, 16)
                        s1[r, sl] = s1[r, sl] + s0[r, sl] * wv
                    for kk in range(PW // 16):
                        lo = lax.bitcast_convert_type(
                            s0[r, pl.ds(kk * 16, 16)], jnp.int32)
                        hi = lax.bitcast_convert_type(
                            s0[r, pl.ds(PW + kk * 16, 16)], jnp.int32)
                        lor = lo + jnp.int32(0x7FFF) + ((lo >> 16) & 1)
                        hir = hi + jnp.int32(0x7FFF) + ((hi >> 16) & 1)
                        g0[r, pl.ds(kk * 16, 16)] = (
                            lax.shift_right_logical(lor, 16)
                            | (hir & _MASK_HI))
                    return 0
                lax.fori_loop(0, ZCH, erow, 0)

                pltpu.sync_copy(s1.at[pl.ds(0, ZCH)], accout_hbm.at[c, pl.ds(r0, ZCH)])
                pltpu.sync_copy(g0.at[pl.ds(0, ZCH)], embout_hbm.at[c, pl.ds(r0, ZCH)])
            return 0
        lax.fori_loop(0, (N_ZCH + NS - 1) // NS, ecopy, 0)

    return run(emb_flat, cols2, rows2, vals2, acc, wvec16)


def kernel(user_emb, item_emb, creator_feat, item_feat, Wc, bc, Wi, bi,
           adj_values, layer_weights, adj_indices):
    rows = adj_indices[0]
    cols = adj_indices[1]
    pad = E_PAD - N_EDGES
    pad_idx = (jnp.arange(pad, dtype=jnp.int32) * 13) % N_NODES
    rows2 = jnp.concatenate([rows, pad_idx]).reshape(NCH_TOT, CHUNK)
    cols_flat = jnp.concatenate([cols, pad_idx]).reshape(NCH_TOT, CHUNK)
    cols2 = jnp.stack([cols_flat, cols_flat + N_NODES], axis=0)
    vals_flat = jnp.concatenate([adj_values, jnp.zeros((pad,), jnp.float32)])
    vals2 = jnp.broadcast_to(
        vals_flat[:, None], (E_PAD, 16)).reshape(NCH_TOT, CHUNK, 16)

    emb_p, acc = _tc_prologue(user_emb, item_emb, creator_feat, item_feat,
                              Wc, bc, Wi, bi, layer_weights)
    for l in range(1, N_LAYERS + 1):
        wvec16 = jnp.broadcast_to(layer_weights[l], (16,))
        emb_p, acc = _sc_layer(emb_p.reshape(NC * N_NODES, PW), cols2, rows2,
                               vals2, acc, wvec16)

    final = jnp.concatenate([acc[0], acc[1]], axis=1)
    return final[:N_USERS], final[N_USERS:]
